# baseline (speedup 1.0000x reference)
import functools

import jax
import jax.numpy as jnp
from jax.experimental import pallas as pl


def _batch_body(nt, scale2, x_ref, w_ref, b_ref, *rest):
    o_ref = rest[-1]
    w = w_ref[...]
    bias = b_ref[0]
    d = w.shape[0]
    xall = x_ref[0]
    for t in range(nt):
        xi = xall[:, t * d:(t + 1) * d]
        xb = xi.astype(jnp.bfloat16)
        xs = xb * jnp.bfloat16(scale2)
        s = jax.lax.dot_general(
            xs, xb, (((1,), (1,)), ((), ())),
            preferred_element_type=jnp.float32)
        s = jnp.maximum(s, 0.0)
        m = jnp.max(s, axis=1, keepdims=True)
        e = jnp.exp2(s - m)
        denom = jnp.sum(e, axis=1, keepdims=True)
        hh = jnp.dot(e.astype(jnp.bfloat16), xb,
                     preferred_element_type=jnp.float32)
        hh = jnp.dot(hh.astype(jnp.bfloat16), w,
                     preferred_element_type=jnp.float32)
        hh = hh / denom
        o_ref[0, :, t, :] = jnp.maximum(hh + bias, 0.0)


def kernel(x, W, b):
    Bx, N, T, D = x.shape
    H = W.shape[1]
    Wb = W.astype(jnp.bfloat16)
    b2 = b.reshape(1, H)
    import math
    scale2 = math.log2(math.e) / math.sqrt(D)
    nc = 4 if Bx % 4 == 0 else 1
    bc = Bx // nc

    body = functools.partial(_batch_body, T, scale2)
    out_shape = jax.ShapeDtypeStruct((Bx, N, T, H), jnp.float32)
    w_spec = pl.BlockSpec((D, H), lambda bb: (0, 0))
    b_spec = pl.BlockSpec((1, H), lambda bb: (0, 0))
    x_spec = pl.BlockSpec((1, N, T * D), lambda bb: (bb, 0, 0))

    acc = None
    for i in range(nc):
        xc = jax.lax.slice_in_dim(x, i * bc, (i + 1) * bc, axis=0)
        x2 = xc.reshape(bc, N, T * D)
        o_spec = pl.BlockSpec(
            (1, N, T, H),
            functools.partial(lambda i0, bb: (i0 + bb, 0, 0, 0), i * bc))
        if acc is None:
            acc = pl.pallas_call(
                body,
                grid=(bc,),
                in_specs=[x_spec, w_spec, b_spec],
                out_specs=o_spec,
                out_shape=out_shape,
            )(x2, Wb, b2)
        else:
            acc = pl.pallas_call(
                body,
                grid=(bc,),
                in_specs=[x_spec, w_spec, b_spec,
                          pl.BlockSpec(memory_space=pl.ANY)],
                out_specs=o_spec,
                out_shape=out_shape,
                input_output_aliases={3: 0},
            )(x2, Wb, b2, acc)
    return acc


# speedup vs baseline: 1.3121x; 1.3121x over previous
import functools
import math

import jax
import jax.numpy as jnp
from jax.experimental import pallas as pl


def _batch_body(nt, scale2, x_ref, w_ref, b_ref, o_ref):
    w = w_ref[...]
    bias = b_ref[0]
    d = w.shape[0]
    xall = x_ref[0]
    for t in range(nt):
        xb = xall[:, t * d:(t + 1) * d]
        xs = xb * jnp.bfloat16(scale2)
        s = jax.lax.dot_general(
            xs, xb, (((1,), (1,)), ((), ())),
            preferred_element_type=jnp.float32)
        s = jnp.maximum(s, 0.0)
        m = jnp.max(s, axis=1, keepdims=True)
        e = jnp.exp2(s - m)
        denom = jnp.sum(e, axis=1, keepdims=True)
        hh = jnp.dot(e.astype(jnp.bfloat16), xb,
                     preferred_element_type=jnp.float32)
        hh = jnp.dot(hh.astype(jnp.bfloat16), w,
                     preferred_element_type=jnp.float32)
        hh = hh / denom
        o_ref[0, :, t, :] = jnp.maximum(hh + bias, 0.0)


def kernel(x, W, b):
    Bx, N, T, D = x.shape
    H = W.shape[1]
    x2 = x.astype(jnp.bfloat16).reshape(Bx, N, T * D)
    Wb = W.astype(jnp.bfloat16)
    b2 = b.reshape(1, H)
    scale2 = math.log2(math.e) / math.sqrt(D)

    out = pl.pallas_call(
        functools.partial(_batch_body, T, scale2),
        grid=(Bx,),
        in_specs=[
            pl.BlockSpec((1, N, T * D), lambda bb: (bb, 0, 0)),
            pl.BlockSpec((D, H), lambda bb: (0, 0)),
            pl.BlockSpec((1, H), lambda bb: (0, 0)),
        ],
        out_specs=pl.BlockSpec((1, N, T, H), lambda bb: (bb, 0, 0, 0)),
        out_shape=jax.ShapeDtypeStruct((Bx, N, T, H), jnp.float32),
    )(x2, Wb, b2)
    return out
